# baseline (speedup 1.0000x reference)
import functools

import jax
import jax.numpy as jnp
from jax.experimental import pallas as pl
from jax.experimental.pallas import tpu as pltpu

NUM_AGENTS = 8
HIDDEN = 1024
NZ = 2048
BATCH = 16384
ROWS = 1024
RB = 256
NSPLIT = HIDDEN // RB
NBSTEPS = NUM_AGENTS * NSPLIT


def _scale_kernel(aidx_ref, b_ref, w_ref, out_ref, parts_ref):
    i = pl.program_id(0)
    parts_ref[i, :] = jnp.max(b_ref[0], axis=0).astype(jnp.float32)

    @pl.when(i == NBSTEPS - 1)
    def _finalize():
        parts = parts_ref[...].reshape(NUM_AGENTS, NSPLIT, NZ)
        masks = (jnp.max(parts, axis=1) > 0).astype(jnp.float32)
        alpha = jnp.sum(masks, axis=0)
        aidx = aidx_ref[0]
        onehot = (jax.lax.broadcasted_iota(jnp.int32, (NUM_AGENTS, 1), 0)
                  == aidx).astype(jnp.float32)
        mask_sel = jnp.sum(masks * onehot, axis=0)
        weights = jnp.zeros((NZ,), jnp.float32)
        for k in range(NUM_AGENTS + 1):
            weights = jnp.where(alpha == float(k), w_ref[0, k], weights)
        out_ref[0, :] = mask_sel * weights


def _mul_kernel(z_ref, s_ref, out_ref):
    out_ref[...] = z_ref[...] * s_ref[...]


@functools.partial(jax.jit, static_argnames=())
def kernel(Z_hat, B, w, agent_idx):
    B3 = B.reshape(NBSTEPS, RB, NZ)
    w2 = w.reshape(1, NUM_AGENTS + 1)
    aidx = jnp.asarray(agent_idx, jnp.int32).reshape((1,))

    scale = pl.pallas_call(
        _scale_kernel,
        grid_spec=pltpu.PrefetchScalarGridSpec(
            num_scalar_prefetch=1,
            grid=(NBSTEPS,),
            in_specs=[
                pl.BlockSpec((1, RB, NZ), lambda i, aidx: (i, 0, 0)),
                pl.BlockSpec((1, NUM_AGENTS + 1), lambda i, aidx: (0, 0)),
            ],
            out_specs=pl.BlockSpec((1, NZ), lambda i, aidx: (0, 0)),
            scratch_shapes=[pltpu.VMEM((NBSTEPS, NZ), jnp.float32)],
        ),
        out_shape=jax.ShapeDtypeStruct((1, NZ), jnp.float32),
    )(aidx, B3, w2)

    out = pl.pallas_call(
        _mul_kernel,
        grid=(BATCH // ROWS,),
        in_specs=[
            pl.BlockSpec((ROWS, NZ), lambda i: (i, 0)),
            pl.BlockSpec((1, NZ), lambda i: (0, 0)),
        ],
        out_specs=pl.BlockSpec((ROWS, NZ), lambda i: (i, 0)),
        out_shape=jax.ShapeDtypeStruct((BATCH, NZ), jnp.float32),
    )(Z_hat, scale)
    return out


# speedup vs baseline: 1.0004x; 1.0004x over previous
import functools

import jax
import jax.numpy as jnp
from jax.experimental import pallas as pl
from jax.experimental.pallas import tpu as pltpu

NUM_AGENTS = 8
HIDDEN = 1024
NZ = 2048
BATCH = 16384
ROWS = 1024
RB = 256
NSPLIT = HIDDEN // RB
NBSTEPS = NUM_AGENTS * NSPLIT


def _scale_kernel(aidx_ref, b_ref, w_ref, out_ref, parts_ref):
    i = pl.program_id(0)
    parts_ref[i, :] = jnp.max(b_ref[0], axis=0).astype(jnp.float32)

    @pl.when(i == NBSTEPS - 1)
    def _finalize():
        parts = parts_ref[...].reshape(NUM_AGENTS, NSPLIT, NZ)
        masks = (jnp.max(parts, axis=1) > 0).astype(jnp.float32)
        alpha = jnp.sum(masks, axis=0)
        aidx = aidx_ref[0]
        onehot = (jax.lax.broadcasted_iota(jnp.int32, (NUM_AGENTS, 1), 0)
                  == aidx).astype(jnp.float32)
        mask_sel = jnp.sum(masks * onehot, axis=0)
        weights = jnp.zeros((NZ,), jnp.float32)
        for k in range(NUM_AGENTS + 1):
            weights = jnp.where(alpha == float(k), w_ref[0, k], weights)
        out_ref[0, :] = mask_sel * weights


def _mul_kernel(z_ref, s_ref, out_ref):
    out_ref[...] = z_ref[...] * s_ref[...]


@functools.partial(jax.jit, static_argnames=())
def kernel(Z_hat, B, w, agent_idx):
    B3 = B.reshape(NBSTEPS, RB, NZ)
    w2 = w.reshape(1, NUM_AGENTS + 1)
    aidx = jnp.asarray(agent_idx, jnp.int32).reshape((1,))

    scale = pl.pallas_call(
        _scale_kernel,
        grid_spec=pltpu.PrefetchScalarGridSpec(
            num_scalar_prefetch=1,
            grid=(NBSTEPS,),
            in_specs=[
                pl.BlockSpec((1, RB, NZ), lambda i, aidx: (i, 0, 0)),
                pl.BlockSpec((1, NUM_AGENTS + 1), lambda i, aidx: (0, 0)),
            ],
            out_specs=pl.BlockSpec((1, NZ), lambda i, aidx: (0, 0)),
            scratch_shapes=[pltpu.VMEM((NBSTEPS, NZ), jnp.float32)],
        ),
        out_shape=jax.ShapeDtypeStruct((1, NZ), jnp.float32),
    )(aidx, B3, w2)

    out = pl.pallas_call(
        _mul_kernel,
        grid=(BATCH // ROWS,),
        in_specs=[
            pl.BlockSpec((ROWS, NZ), lambda i: (i, 0)),
            pl.BlockSpec((1, NZ), lambda i: (0, 0)),
        ],
        out_specs=pl.BlockSpec((ROWS, NZ), lambda i: (i, 0)),
        out_shape=jax.ShapeDtypeStruct((BATCH, NZ), jnp.float32),
        compiler_params=pltpu.CompilerParams(
            dimension_semantics=("parallel",)),
    )(Z_hat, scale)
    return out
